# Initial kernel scaffold; baseline (speedup 1.0000x reference)
import jax
import jax.numpy as jnp
from jax.experimental import pallas as pl


def kernel(indices, table, W1, b1, W2, b2):
    raise NotImplementedError("write your pallas kernel here")


# speedup vs baseline: 5.8528x; 5.8528x over previous
import functools

import jax
import jax.numpy as jnp
from jax import lax
from jax.experimental import pallas as pl
from jax.experimental.pallas import tpu as pltpu
from jax.experimental.pallas import tpu_sc as plsc

_B, _L, _D, _H = 16384, 200, 32, 64
_C = 104
_LP = 2 * _C

_info = plsc.get_sparse_core_info()
_NC, _NS = _info.num_cores, _info.num_subcores
_NW = _NC * _NS
_RPW = _B // _NW
_G = 16
_NGROUPS = _RPW // _G


def _pool_body(idx_hbm, table_hbm, out_hbm, idx_v, buf0, buf1, out_v, sem0, sem1):
    wid = lax.axis_index("s") * _NC + lax.axis_index("c")
    base = wid * _RPW

    def accumulate(buf):
        z = jnp.zeros((16,), jnp.float32)

        def body(i, accs):
            a0, b0, a1, b1 = accs
            r = i * 4
            a0 = a0 + buf[r, pl.ds(0, 16)]
            a1 = a1 + buf[r, pl.ds(16, 16)]
            b0 = b0 + buf[r + 1, pl.ds(0, 16)]
            b1 = b1 + buf[r + 1, pl.ds(16, 16)]
            a0 = a0 + buf[r + 2, pl.ds(0, 16)]
            a1 = a1 + buf[r + 2, pl.ds(16, 16)]
            b0 = b0 + buf[r + 3, pl.ds(0, 16)]
            b1 = b1 + buf[r + 3, pl.ds(16, 16)]
            return (a0, b0, a1, b1)

        a0, b0, a1, b1 = lax.fori_loop(0, _LP // 4, body, (z, z, z, z))
        return a0 + b0, a1 + b1

    def start(r, buf, sem):
        c0 = pltpu.async_copy(table_hbm.at[idx_v.at[2 * r]], buf.at[pl.ds(0, _C)], sem)
        c1 = pltpu.async_copy(table_hbm.at[idx_v.at[2 * r + 1]], buf.at[pl.ds(_C, _C)], sem)
        return c0, c1

    def group(g, carry):
        row0 = base + g * _G
        pltpu.sync_copy(idx_hbm.at[pl.ds(2 * row0, 2 * _G)], idx_v)
        cps = start(0, buf0, sem0)
        for r in range(_G):
            cur_buf = buf0 if r % 2 == 0 else buf1
            nxt_buf = buf1 if r % 2 == 0 else buf0
            nxt_sem = sem1 if r % 2 == 0 else sem0
            nxt = start(r + 1, nxt_buf, nxt_sem) if r + 1 < _G else None
            cps[0].wait()
            cps[1].wait()
            lo, hi = accumulate(cur_buf)
            out_v[r, pl.ds(0, 16)] = lo
            out_v[r, pl.ds(16, 16)] = hi
            cps = nxt
        pltpu.sync_copy(out_v, out_hbm.at[pl.ds(row0, _G)])
        return carry

    lax.fori_loop(0, _NGROUPS, group, 0)


_pool = functools.partial(
    pl.kernel,
    out_type=jax.ShapeDtypeStruct((_B, _D), jnp.float32),
    mesh=plsc.VectorSubcoreMesh(core_axis_name="c", subcore_axis_name="s"),
    scratch_types=[
        pltpu.VMEM((2 * _G, _C), jnp.int32),
        pltpu.VMEM((_LP, _D), jnp.float32),
        pltpu.VMEM((_LP, _D), jnp.float32),
        pltpu.VMEM((_G, _D), jnp.float32),
        pltpu.SemaphoreType.DMA,
        pltpu.SemaphoreType.DMA,
    ],
    compiler_params=pltpu.CompilerParams(use_tc_tiling_on_sc=False),
)(_pool_body)


_BB = 2048


def _mlp_body(idx_ref, ps_ref, w1_ref, b1_ref, w2_ref, b2_ref, out_ref):
    cnt = jnp.sum((idx_ref[...] != 0).astype(jnp.float32), axis=1, keepdims=True)
    denom = jnp.maximum(cnt, 1.0)
    pooled = ps_ref[...] / denom
    h = lax.dot_general(pooled, w1_ref[...], (((1,), (1,)), ((), ())),
                        preferred_element_type=jnp.float32)
    h = jnp.maximum(h + b1_ref[...], 0.0)
    o = jnp.sum(h * w2_ref[...], axis=1, keepdims=True)
    out_ref[...] = o + b2_ref[0, 0]


_mlp = pl.pallas_call(
    _mlp_body,
    grid=(_B // _BB,),
    in_specs=[
        pl.BlockSpec((_BB, _L), lambda i: (i, 0)),
        pl.BlockSpec((_BB, _D), lambda i: (i, 0)),
        pl.BlockSpec((_H, _D), lambda i: (0, 0)),
        pl.BlockSpec((1, _H), lambda i: (0, 0)),
        pl.BlockSpec((1, _H), lambda i: (0, 0)),
        pl.BlockSpec(memory_space=pltpu.SMEM),
    ],
    out_specs=pl.BlockSpec((_BB, 1), lambda i: (i, 0)),
    out_shape=jax.ShapeDtypeStruct((_B, 1), jnp.float32),
)


def kernel(indices, table, W1, b1, W2, b2):
    idx = indices.astype(jnp.int32)
    idx_pad = jnp.pad(idx, ((0, 0), (0, _LP - _L))).reshape(2 * _B, _C)
    pooled_sum = _pool(idx_pad, table)
    out = _mlp(idx, pooled_sum, W1, b1.reshape(1, _H), W2, b2.reshape(1, 1))
    return out.reshape(_B)


# speedup vs baseline: 12.1364x; 2.0736x over previous
import functools

import jax
import jax.numpy as jnp
from jax import lax
from jax.experimental import pallas as pl
from jax.experimental.pallas import tpu as pltpu
from jax.experimental.pallas import tpu_sc as plsc

_B, _L, _D, _H = 16384, 200, 32, 64
_C0, _C1 = 96, 104

_info = plsc.get_sparse_core_info()
_NC, _NS = _info.num_cores, _info.num_subcores
_NW = _NC * _NS
_RPW = _B // _NW
_G = 16
_NGROUPS = _RPW // _G


def _pool_body(idx_hbm, table_hbm, out_hbm, idx_v, buf0, buf1, out_v, sem0, sem1):
    wid = lax.axis_index("s") * _NC + lax.axis_index("c")
    base = wid * _RPW

    def issue(r, buf, sem):
        pltpu.async_copy(table_hbm.at[idx_v.at[r, pl.ds(0, _C0)]],
                         buf.at[pl.ds(0, _C0)], sem)
        pltpu.async_copy(table_hbm.at[idx_v.at[r, pl.ds(_C0, _C1)]],
                         buf.at[pl.ds(_C0, _C1)], sem)

    def wait_full(buf, sem):
        pltpu.make_async_copy(table_hbm.at[pl.ds(0, _L)], buf, sem).wait()

    def acc_store(buf, r):
        a = [jnp.zeros((16,), jnp.float32) for _ in range(4)]
        b = [jnp.zeros((16,), jnp.float32) for _ in range(4)]
        for rr in range(_L):
            k = rr % 4
            a[k] = a[k] + buf[rr, pl.ds(0, 16)]
            b[k] = b[k] + buf[rr, pl.ds(16, 16)]
        out_v[r, pl.ds(0, 16)] = (a[0] + a[1]) + (a[2] + a[3])
        out_v[r, pl.ds(16, 16)] = (b[0] + b[1]) + (b[2] + b[3])

    def pair(p, carry):
        r0 = 2 * p
        issue(r0 + 1, buf1, sem1)
        wait_full(buf0, sem0)
        acc_store(buf0, r0)

        @pl.when(p + 1 < _G // 2)
        def _():
            issue(r0 + 2, buf0, sem0)

        wait_full(buf1, sem1)
        acc_store(buf1, r0 + 1)
        return carry

    def group(g, carry):
        row0 = base + g * _G
        pltpu.sync_copy(idx_hbm.at[pl.ds(row0, _G)], idx_v)
        issue(0, buf0, sem0)
        lax.fori_loop(0, _G // 2, pair, 0)
        pltpu.sync_copy(out_v, out_hbm.at[pl.ds(row0, _G)])
        return carry

    lax.fori_loop(0, _NGROUPS, group, 0)


_pool = functools.partial(
    pl.kernel,
    out_type=jax.ShapeDtypeStruct((_B, _D), jnp.float32),
    mesh=plsc.VectorSubcoreMesh(core_axis_name="c", subcore_axis_name="s"),
    scratch_types=[
        pltpu.VMEM((_G, _L), jnp.int32),
        pltpu.VMEM((_L, _D), jnp.float32),
        pltpu.VMEM((_L, _D), jnp.float32),
        pltpu.VMEM((_G, _D), jnp.float32),
        pltpu.SemaphoreType.DMA,
        pltpu.SemaphoreType.DMA,
    ],
    compiler_params=pltpu.CompilerParams(use_tc_tiling_on_sc=False),
)(_pool_body)


_BB = 2048


def _mlp_body(idx_ref, ps_ref, w1_ref, b1_ref, w2_ref, b2_ref, out_ref):
    cnt = jnp.sum((idx_ref[...] != 0).astype(jnp.float32), axis=1, keepdims=True)
    denom = jnp.maximum(cnt, 1.0)
    pooled = ps_ref[...] / denom
    h = lax.dot_general(pooled, w1_ref[...], (((1,), (1,)), ((), ())),
                        preferred_element_type=jnp.float32)
    h = jnp.maximum(h + b1_ref[...], 0.0)
    o = jnp.sum(h * w2_ref[...], axis=1, keepdims=True)
    out_ref[...] = o + b2_ref[0, 0]


_mlp = pl.pallas_call(
    _mlp_body,
    grid=(_B // _BB,),
    in_specs=[
        pl.BlockSpec((_BB, _L), lambda i: (i, 0)),
        pl.BlockSpec((_BB, _D), lambda i: (i, 0)),
        pl.BlockSpec((_H, _D), lambda i: (0, 0)),
        pl.BlockSpec((1, _H), lambda i: (0, 0)),
        pl.BlockSpec((1, _H), lambda i: (0, 0)),
        pl.BlockSpec(memory_space=pltpu.SMEM),
    ],
    out_specs=pl.BlockSpec((_BB, 1), lambda i: (i, 0)),
    out_shape=jax.ShapeDtypeStruct((_B, 1), jnp.float32),
)


def kernel(indices, table, W1, b1, W2, b2):
    idx = indices.astype(jnp.int32)
    pooled_sum = _pool(idx, table)
    out = _mlp(idx, pooled_sum, W1, b1.reshape(1, _H), W2, b2.reshape(1, 1))
    return out.reshape(_B)
